# baseline (speedup 1.0000x reference)
import functools

import jax
import jax.numpy as jnp
from jax import lax
from jax.experimental import pallas as pl
from jax.experimental.pallas import tpu as pltpu
from jax.experimental.pallas import tpu_sc as plsc

N = 10000
NP = 10240
E = 640000
EP = 655360
D = 128
G = 64
C = 10
PAD_EDGES = float(EP - E)

SB = 5
TSB = EP // (SB * 128)
NSB0 = 17
NSB1 = 47
ROWS_PT = NP // 16


def _sc_agg_body(x_hbm, sd3, w3, zeros2d, zeros1d,
                 agg_out, cnt_out,
                 agg_sh, hist, sd_v, w_v, rows0, rows1,
                 sg0, sg1, ss0, ss1):
    c = lax.axis_index("c")
    s = lax.axis_index("s")
    base_sb = jnp.where(c == 0, s * NSB0, 16 * NSB0 + s * NSB1)
    nsb = jnp.where(c == 0, NSB0, NSB1)
    r = s * ROWS_PT
    pltpu.sync_copy(zeros2d.at[pl.ds(r, ROWS_PT)], agg_sh.at[pl.ds(r, ROWS_PT)])
    pltpu.sync_copy(zeros1d, hist)
    plsc.subcore_barrier()

    bufs = (rows0, rows1)
    gsems = (sg0, sg1)
    ssems = (ss0, ss1)
    ones16 = jnp.ones((16,), jnp.float32)

    def superchunk(k, carry):
        g = base_sb + k
        pltpu.sync_copy(sd3.at[g], sd_v)
        pltpu.sync_copy(w3.at[g], w_v)

        def gather(j):
            return pltpu.async_copy(x_hbm.at[sd_v.at[j]], bufs[j % 2],
                                    gsems[j % 2])

        def scale(j):
            buf = bufs[j % 2]

            def body(g, c2):
                wv = w_v[pl.ds(j * 128 + g * 16, 16)]
                dst16 = sd_v[SB + j, pl.ds(g * 16, 16)]
                plsc.addupdate_scatter(hist, [dst16], ones16)
                for u in range(16):
                    ws = wv[u]
                    e = g * 16 + u
                    for f4 in range(D // 16):
                        sl = buf[e, pl.ds(f4 * 16, 16)]
                        buf[e, pl.ds(f4 * 16, 16)] = sl * ws
                return c2

            lax.fori_loop(0, 8, body, 0)

        def scatter(j):
            return pltpu.async_copy(bufs[j % 2], agg_sh.at[sd_v.at[SB + j]],
                                    ssems[j % 2], add=True)

        gh = {0: gather(0)}
        sh = {}
        for j in range(SB):
            gh[j].wait()
            if j + 1 < SB:
                if j >= 1:
                    sh[j - 1].wait()
                gh[j + 1] = gather(j + 1)
            scale(j)
            sh[j] = scatter(j)
        sh[SB - 2].wait()
        sh[SB - 1].wait()
        return carry

    lax.fori_loop(0, nsb, superchunk, 0)
    plsc.subcore_barrier()
    pltpu.sync_copy(agg_sh.at[pl.ds(r, ROWS_PT)],
                    agg_out.at[c, pl.ds(r, ROWS_PT)])
    pltpu.sync_copy(hist, cnt_out.at[c, s])


_sc_agg = functools.partial(
    pl.kernel,
    mesh=plsc.VectorSubcoreMesh(core_axis_name="c", subcore_axis_name="s"),
    compiler_params=pltpu.CompilerParams(needs_layout_passes=False),
    out_type=(
        jax.ShapeDtypeStruct((2, NP, D), jnp.float32),
        jax.ShapeDtypeStruct((2, 16, NP), jnp.float32),
    ),
    scratch_types=[
        pltpu.VMEM_SHARED((NP, D), jnp.float32),
        pltpu.VMEM((NP,), jnp.float32),
        pltpu.VMEM((2 * SB, 128), jnp.int32),
        pltpu.VMEM((SB * 128,), jnp.float32),
        pltpu.VMEM((128, D), jnp.float32),
        pltpu.VMEM((128, D), jnp.float32),
        pltpu.SemaphoreType.DMA,
        pltpu.SemaphoreType.DMA,
        pltpu.SemaphoreType.DMA,
        pltpu.SemaphoreType.DMA,
    ],
)(_sc_agg_body)


def _mm(a, b):
    return jnp.dot(a, b, precision="highest", preferred_element_type=jnp.float32)


BR = 1024


def _tc1_body(agg_ref, cnt_ref, x_ref, wl_ref, wr_ref, b_ref, out_ref):
    i = pl.program_id(0)
    aggf = agg_ref[0] + agg_ref[1]
    cnt = jnp.sum(cnt_ref[...], axis=0)
    rows = i * BR + lax.broadcasted_iota(jnp.int32, (BR, 1), 0)
    cnt = cnt - jnp.where(rows == 0, PAD_EDGES, 0.0)
    inv = 1.0 / jnp.maximum(cnt, 1.0)
    h = _mm(aggf * inv, wl_ref[...]) + _mm(x_ref[...], wr_ref[...]) + b_ref[...]
    out_ref[...] = jnp.maximum(h, 0.0)


_tc1 = pl.pallas_call(
    _tc1_body,
    grid=(NP // BR,),
    in_specs=[
        pl.BlockSpec((2, BR, D), lambda i: (0, i, 0)),
        pl.BlockSpec((32, BR, 1), lambda i: (0, i, 0)),
        pl.BlockSpec((BR, D), lambda i: (i, 0)),
        pl.BlockSpec((D, D), lambda i: (0, 0)),
        pl.BlockSpec((D, D), lambda i: (0, 0)),
        pl.BlockSpec((1, D), lambda i: (0, 0)),
    ],
    out_specs=pl.BlockSpec((BR, D), lambda i: (i, 0)),
    out_shape=jax.ShapeDtypeStruct((NP, D), jnp.float32),
)


def _tc2_body(agg_ref, cnt_ref, h1_ref, batch_ref, wl_ref, wr_ref, b_ref,
              l1w_ref, l1b_ref, l2w_ref, l2b_ref, out_ref,
              pool_scr, cnt_scr):
    i = pl.program_id(0)
    aggf = agg_ref[0] + agg_ref[1]
    cnt = jnp.sum(cnt_ref[...], axis=0)
    rows = i * BR + lax.broadcasted_iota(jnp.int32, (BR, 1), 0)
    cnt = cnt - jnp.where(rows == 0, PAD_EDGES, 0.0)
    inv = 1.0 / jnp.maximum(cnt, 1.0)
    h2 = _mm(aggf * inv, wl_ref[...]) + _mm(h1_ref[...], wr_ref[...]) + b_ref[...]
    h2 = jnp.maximum(h2, 0.0)

    batch_blk = batch_ref[...]
    oh = (batch_blk == lax.broadcasted_iota(jnp.int32, (BR, G), 1))
    oh = oh.astype(jnp.float32)
    part = lax.dot_general(oh, h2, (((0,), (0,)), ((), ())),
                           precision="highest",
                           preferred_element_type=jnp.float32)
    pcnt = jnp.broadcast_to(jnp.sum(oh, axis=0)[:, None], (G, D))

    @pl.when(i == 0)
    def _():
        pool_scr[...] = jnp.zeros((G, D), jnp.float32)
        cnt_scr[...] = jnp.zeros((G, D), jnp.float32)

    pool_scr[...] += part
    cnt_scr[...] += pcnt

    @pl.when(i == NP // BR - 1)
    def _():
        pooled = pool_scr[...] / jnp.maximum(cnt_scr[...], 1.0)
        t = jnp.maximum(_mm(pooled, l1w_ref[...]) + l1b_ref[...], 0.0)
        logits = _mm(t, l2w_ref[...]) + l2b_ref[...]
        m = jnp.max(logits, axis=1, keepdims=True)
        lse = jnp.log(jnp.sum(jnp.exp(logits - m), axis=1, keepdims=True)) + m
        out_ref[...] = logits - lse


_tc2 = pl.pallas_call(
    _tc2_body,
    grid=(NP // BR,),
    in_specs=[
        pl.BlockSpec((2, BR, D), lambda i: (0, i, 0)),
        pl.BlockSpec((32, BR, 1), lambda i: (0, i, 0)),
        pl.BlockSpec((BR, D), lambda i: (i, 0)),
        pl.BlockSpec((BR, 1), lambda i: (i, 0)),
        pl.BlockSpec((D, D), lambda i: (0, 0)),
        pl.BlockSpec((D, D), lambda i: (0, 0)),
        pl.BlockSpec((1, D), lambda i: (0, 0)),
        pl.BlockSpec((D, D), lambda i: (0, 0)),
        pl.BlockSpec((1, D), lambda i: (0, 0)),
        pl.BlockSpec((D, C), lambda i: (0, 0)),
        pl.BlockSpec((1, C), lambda i: (0, 0)),
    ],
    out_specs=pl.BlockSpec((G, C), lambda i: (0, 0)),
    out_shape=jax.ShapeDtypeStruct((G, C), jnp.float32),
    scratch_shapes=[
        pltpu.VMEM((G, D), jnp.float32),
        pltpu.VMEM((G, D), jnp.float32),
    ],
)


@jax.jit
def kernel(x, edge_index, edge_weight, batch,
           conv1_Wl, conv1_Wr, conv1_b,
           conv2_Wl, conv2_Wr, conv2_b,
           lin1_W, lin1_b, lin2_W, lin2_b):
    src = edge_index[0].astype(jnp.int32)
    dst = edge_index[1].astype(jnp.int32)
    w = edge_weight.astype(jnp.float32)
    pad = EP - E
    spad = jnp.concatenate([src, jnp.zeros((pad,), jnp.int32)])
    dpad = jnp.concatenate([dst, jnp.zeros((pad,), jnp.int32)])
    srcr = spad.reshape(TSB, SB, 128)
    dstr = dpad.reshape(TSB, SB, 128)
    sd3 = jnp.concatenate([srcr, dstr], axis=1)
    w3 = jnp.concatenate([w, jnp.zeros((pad,), jnp.float32)]).reshape(
        TSB, SB * 128)

    xp = jnp.pad(x, ((0, NP - N), (0, 0)))
    zeros2d = jnp.zeros((NP, D), jnp.float32)
    zeros1d = jnp.zeros((NP,), jnp.float32)

    agg1, cnt1 = _sc_agg(xp, sd3, w3, zeros2d, zeros1d)
    cnt1r = cnt1.reshape(32, NP, 1)
    h1 = _tc1(agg1, cnt1r, xp, conv1_Wl, conv1_Wr, conv1_b.reshape(1, D))
    agg2, _ = _sc_agg(h1, sd3, w3, zeros2d, zeros1d)

    batch_p = jnp.concatenate(
        [batch.astype(jnp.int32), jnp.full((NP - N,), G, jnp.int32)]
    ).reshape(NP, 1)
    out = _tc2(agg2, cnt1r, h1, batch_p,
               conv2_Wl, conv2_Wr, conv2_b.reshape(1, D),
               lin1_W, lin1_b.reshape(1, D),
               lin2_W, lin2_b.reshape(1, C))
    return out


# speedup vs baseline: 1.2379x; 1.2379x over previous
import functools

import jax
import jax.numpy as jnp
from jax import lax
from jax.experimental import pallas as pl
from jax.experimental.pallas import tpu as pltpu
from jax.experimental.pallas import tpu_sc as plsc

N = 10000
NP = 10240
E = 640000
EP = 655360
D = 128
G = 64
C = 10
PAD_EDGES = float(EP - E)

SB = 5
TSB = EP // (SB * 128)
NSB0 = 45
NSB1 = 19
ROWS_PT = NP // 16


def _sc_agg_body(x_hbm, sd3, w3, zeros2d, zeros1d,
                 agg_out, cnt_out,
                 agg_sh, hist, sd_v, w_v, rows0, rows1,
                 sg0, sg1, ss0, ss1):
    c = lax.axis_index("c")
    s = lax.axis_index("s")
    base_sb = jnp.where(c == 0, s * NSB0, 16 * NSB0 + s * NSB1)
    nsb = jnp.where(c == 0, NSB0, NSB1)
    r = s * ROWS_PT
    pltpu.sync_copy(zeros2d.at[pl.ds(r, ROWS_PT)], agg_sh.at[pl.ds(r, ROWS_PT)])
    pltpu.sync_copy(zeros1d, hist)
    plsc.subcore_barrier()

    bufs = (rows0, rows1)
    gsems = (sg0, sg1)
    ssems = (ss0, ss1)
    ones16 = jnp.ones((16,), jnp.float32)

    def superchunk(k, carry):
        g = base_sb + k
        pltpu.sync_copy(sd3.at[g], sd_v)
        pltpu.sync_copy(w3.at[g], w_v)

        def gather(j):
            return pltpu.async_copy(x_hbm.at[sd_v.at[j]], bufs[j % 2],
                                    gsems[j % 2])

        def scale(j):
            buf = bufs[j % 2]

            def body(g, c2):
                wv = w_v[pl.ds(j * 128 + g * 16, 16)]
                dst16 = sd_v[SB + j, pl.ds(g * 16, 16)]
                plsc.addupdate_scatter(hist, [dst16], ones16)
                for u in range(16):
                    ws = wv[u]
                    e = g * 16 + u
                    for f4 in range(D // 16):
                        sl = buf[e, pl.ds(f4 * 16, 16)]
                        buf[e, pl.ds(f4 * 16, 16)] = sl * ws
                return c2

            lax.fori_loop(0, 8, body, 0)

        def scatter(j):
            return pltpu.async_copy(bufs[j % 2], agg_sh.at[sd_v.at[SB + j]],
                                    ssems[j % 2], add=True)

        gh = {0: gather(0)}
        sh = {}
        for j in range(SB):
            gh[j].wait()
            if j + 1 < SB:
                if j >= 1:
                    sh[j - 1].wait()
                gh[j + 1] = gather(j + 1)
            scale(j)
            sh[j] = scatter(j)
        sh[SB - 2].wait()
        sh[SB - 1].wait()
        return carry

    lax.fori_loop(0, nsb, superchunk, 0)
    plsc.subcore_barrier()
    pltpu.sync_copy(agg_sh.at[pl.ds(r, ROWS_PT)],
                    agg_out.at[c, pl.ds(r, ROWS_PT)])
    pltpu.sync_copy(hist, cnt_out.at[c, s])


_sc_agg = functools.partial(
    pl.kernel,
    mesh=plsc.VectorSubcoreMesh(core_axis_name="c", subcore_axis_name="s"),
    compiler_params=pltpu.CompilerParams(needs_layout_passes=False),
    out_type=(
        jax.ShapeDtypeStruct((2, NP, D), jnp.float32),
        jax.ShapeDtypeStruct((2, 16, NP), jnp.float32),
    ),
    scratch_types=[
        pltpu.VMEM_SHARED((NP, D), jnp.float32),
        pltpu.VMEM((NP,), jnp.float32),
        pltpu.VMEM((2 * SB, 128), jnp.int32),
        pltpu.VMEM((SB * 128,), jnp.float32),
        pltpu.VMEM((128, D), jnp.float32),
        pltpu.VMEM((128, D), jnp.float32),
        pltpu.SemaphoreType.DMA,
        pltpu.SemaphoreType.DMA,
        pltpu.SemaphoreType.DMA,
        pltpu.SemaphoreType.DMA,
    ],
)(_sc_agg_body)


def _mm(a, b):
    return jnp.dot(a, b, precision="highest", preferred_element_type=jnp.float32)


BR = 1024


def _tc1_body(agg_ref, cnt_ref, x_ref, wl_ref, wr_ref, b_ref, out_ref):
    i = pl.program_id(0)
    aggf = agg_ref[0] + agg_ref[1]
    cnt = jnp.sum(cnt_ref[...], axis=0)
    rows = i * BR + lax.broadcasted_iota(jnp.int32, (BR, 1), 0)
    cnt = cnt - jnp.where(rows == 0, PAD_EDGES, 0.0)
    inv = 1.0 / jnp.maximum(cnt, 1.0)
    h = _mm(aggf * inv, wl_ref[...]) + _mm(x_ref[...], wr_ref[...]) + b_ref[...]
    out_ref[...] = jnp.maximum(h, 0.0)


_tc1 = pl.pallas_call(
    _tc1_body,
    grid=(NP // BR,),
    in_specs=[
        pl.BlockSpec((2, BR, D), lambda i: (0, i, 0)),
        pl.BlockSpec((32, BR, 1), lambda i: (0, i, 0)),
        pl.BlockSpec((BR, D), lambda i: (i, 0)),
        pl.BlockSpec((D, D), lambda i: (0, 0)),
        pl.BlockSpec((D, D), lambda i: (0, 0)),
        pl.BlockSpec((1, D), lambda i: (0, 0)),
    ],
    out_specs=pl.BlockSpec((BR, D), lambda i: (i, 0)),
    out_shape=jax.ShapeDtypeStruct((NP, D), jnp.float32),
)


def _tc2_body(agg_ref, cnt_ref, h1_ref, batch_ref, wl_ref, wr_ref, b_ref,
              l1w_ref, l1b_ref, l2w_ref, l2b_ref, out_ref,
              pool_scr, cnt_scr):
    i = pl.program_id(0)
    aggf = agg_ref[0] + agg_ref[1]
    cnt = jnp.sum(cnt_ref[...], axis=0)
    rows = i * BR + lax.broadcasted_iota(jnp.int32, (BR, 1), 0)
    cnt = cnt - jnp.where(rows == 0, PAD_EDGES, 0.0)
    inv = 1.0 / jnp.maximum(cnt, 1.0)
    h2 = _mm(aggf * inv, wl_ref[...]) + _mm(h1_ref[...], wr_ref[...]) + b_ref[...]
    h2 = jnp.maximum(h2, 0.0)

    batch_blk = batch_ref[...]
    oh = (batch_blk == lax.broadcasted_iota(jnp.int32, (BR, G), 1))
    oh = oh.astype(jnp.float32)
    part = lax.dot_general(oh, h2, (((0,), (0,)), ((), ())),
                           precision="highest",
                           preferred_element_type=jnp.float32)
    pcnt = jnp.broadcast_to(jnp.sum(oh, axis=0)[:, None], (G, D))

    @pl.when(i == 0)
    def _():
        pool_scr[...] = jnp.zeros((G, D), jnp.float32)
        cnt_scr[...] = jnp.zeros((G, D), jnp.float32)

    pool_scr[...] += part
    cnt_scr[...] += pcnt

    @pl.when(i == NP // BR - 1)
    def _():
        pooled = pool_scr[...] / jnp.maximum(cnt_scr[...], 1.0)
        t = jnp.maximum(_mm(pooled, l1w_ref[...]) + l1b_ref[...], 0.0)
        logits = _mm(t, l2w_ref[...]) + l2b_ref[...]
        m = jnp.max(logits, axis=1, keepdims=True)
        lse = jnp.log(jnp.sum(jnp.exp(logits - m), axis=1, keepdims=True)) + m
        out_ref[...] = logits - lse


_tc2 = pl.pallas_call(
    _tc2_body,
    grid=(NP // BR,),
    in_specs=[
        pl.BlockSpec((2, BR, D), lambda i: (0, i, 0)),
        pl.BlockSpec((32, BR, 1), lambda i: (0, i, 0)),
        pl.BlockSpec((BR, D), lambda i: (i, 0)),
        pl.BlockSpec((BR, 1), lambda i: (i, 0)),
        pl.BlockSpec((D, D), lambda i: (0, 0)),
        pl.BlockSpec((D, D), lambda i: (0, 0)),
        pl.BlockSpec((1, D), lambda i: (0, 0)),
        pl.BlockSpec((D, D), lambda i: (0, 0)),
        pl.BlockSpec((1, D), lambda i: (0, 0)),
        pl.BlockSpec((D, C), lambda i: (0, 0)),
        pl.BlockSpec((1, C), lambda i: (0, 0)),
    ],
    out_specs=pl.BlockSpec((G, C), lambda i: (0, 0)),
    out_shape=jax.ShapeDtypeStruct((G, C), jnp.float32),
    scratch_shapes=[
        pltpu.VMEM((G, D), jnp.float32),
        pltpu.VMEM((G, D), jnp.float32),
    ],
)


@jax.jit
def kernel(x, edge_index, edge_weight, batch,
           conv1_Wl, conv1_Wr, conv1_b,
           conv2_Wl, conv2_Wr, conv2_b,
           lin1_W, lin1_b, lin2_W, lin2_b):
    src = edge_index[0].astype(jnp.int32)
    dst = edge_index[1].astype(jnp.int32)
    w = edge_weight.astype(jnp.float32)
    pad = EP - E
    spad = jnp.concatenate([src, jnp.zeros((pad,), jnp.int32)])
    dpad = jnp.concatenate([dst, jnp.zeros((pad,), jnp.int32)])
    srcr = spad.reshape(TSB, SB, 128)
    dstr = dpad.reshape(TSB, SB, 128)
    sd3 = jnp.concatenate([srcr, dstr], axis=1)
    w3 = jnp.concatenate([w, jnp.zeros((pad,), jnp.float32)]).reshape(
        TSB, SB * 128)

    xp = jnp.pad(x, ((0, NP - N), (0, 0)))
    zeros2d = jnp.zeros((NP, D), jnp.float32)
    zeros1d = jnp.zeros((NP,), jnp.float32)

    agg1, cnt1 = _sc_agg(xp, sd3, w3, zeros2d, zeros1d)
    cnt1r = cnt1.reshape(32, NP, 1)
    h1 = _tc1(agg1, cnt1r, xp, conv1_Wl, conv1_Wr, conv1_b.reshape(1, D))
    agg2, _ = _sc_agg(h1, sd3, w3, zeros2d, zeros1d)

    batch_p = jnp.concatenate(
        [batch.astype(jnp.int32), jnp.full((NP - N,), G, jnp.int32)]
    ).reshape(NP, 1)
    out = _tc2(agg2, cnt1r, h1, batch_p,
               conv2_Wl, conv2_Wr, conv2_b.reshape(1, D),
               lin1_W, lin1_b.reshape(1, D),
               lin2_W, lin2_b.reshape(1, C))
    return out


# speedup vs baseline: 2.6834x; 2.1677x over previous
import functools

import jax
import jax.numpy as jnp
from jax import lax
from jax.experimental import pallas as pl
from jax.experimental.pallas import tpu as pltpu
from jax.experimental.pallas import tpu_sc as plsc

N = 10000
NP = 10240
E = 640000
EP = 655360
D = 128
G = 64
C = 10
PAD_EDGES = float(EP - E)

SB = 5
TSB = EP // (SB * 128)
NSB0 = 32
NSB1 = 32
ROWS_PT = NP // 16


def _sc_agg_body(x_hbm, sd3, w3, zeros2d, zeros1d,
                 agg_out, cnt_out,
                 agg_sh, hist, sd_v, w_v, rows0, rows1,
                 sg0, sg1, ss0, ss1):
    c = lax.axis_index("c")
    s = lax.axis_index("s")
    base_sb = jnp.where(c == 0, s * NSB0, 16 * NSB0 + s * NSB1)
    nsb = jnp.where(c == 0, NSB0, NSB1)
    r = s * ROWS_PT
    pltpu.sync_copy(zeros2d.at[pl.ds(r, ROWS_PT)], agg_sh.at[pl.ds(r, ROWS_PT)])
    pltpu.sync_copy(zeros1d, hist)
    plsc.subcore_barrier()

    bufs = (rows0, rows1)
    gsems = (sg0, sg1)
    ssems = (ss0, ss1)
    ones16 = jnp.ones((16,), jnp.float32)

    def superchunk(k, carry):
        g = base_sb + k
        pltpu.sync_copy(sd3.at[g], sd_v)
        pltpu.sync_copy(w3.at[g], w_v)

        def gather(j):
            return pltpu.async_copy(x_hbm.at[sd_v.at[j]], bufs[j % 2],
                                    gsems[j % 2])

        def scale(j):
            buf = bufs[j % 2]

            def body(g, c2):
                wv = w_v[pl.ds(j * 128 + g * 16, 16)]
                dst16 = sd_v[SB + j, pl.ds(g * 16, 16)]
                plsc.addupdate_scatter(hist, [dst16], ones16)
                for u in range(16):
                    ws = wv[u]
                    e = g * 16 + u
                    for f4 in range(D // 16):
                        sl = buf[e, pl.ds(f4 * 16, 16)]
                        buf[e, pl.ds(f4 * 16, 16)] = sl * ws
                return c2

            lax.fori_loop(0, 8, body, 0)

        def scatter(j):
            return pltpu.async_copy(bufs[j % 2], agg_sh.at[sd_v.at[SB + j]],
                                    ssems[j % 2], add=True)

        gh = {0: gather(0)}
        sh = {}
        for j in range(SB):
            gh[j].wait()
            if j + 1 < SB:
                if j >= 1:
                    sh[j - 1].wait()
                gh[j + 1] = gather(j + 1)
            scale(j)
            sh[j] = scatter(j)
        sh[SB - 2].wait()
        sh[SB - 1].wait()
        return carry

    lax.fori_loop(0, nsb, superchunk, 0)
    plsc.subcore_barrier()
    pltpu.sync_copy(agg_sh.at[pl.ds(r, ROWS_PT)],
                    agg_out.at[c, pl.ds(r, ROWS_PT)])
    pltpu.sync_copy(hist, cnt_out.at[c, s])


_sc_agg = functools.partial(
    pl.kernel,
    mesh=plsc.VectorSubcoreMesh(core_axis_name="c", subcore_axis_name="s"),
    compiler_params=pltpu.CompilerParams(needs_layout_passes=False),
    out_type=(
        jax.ShapeDtypeStruct((2, NP, D), jnp.float32),
        jax.ShapeDtypeStruct((2, 16, NP), jnp.float32),
    ),
    scratch_types=[
        pltpu.VMEM_SHARED((NP, D), jnp.float32),
        pltpu.VMEM((NP,), jnp.float32),
        pltpu.VMEM((2 * SB, 128), jnp.int32),
        pltpu.VMEM((SB * 128,), jnp.float32),
        pltpu.VMEM((128, D), jnp.float32),
        pltpu.VMEM((128, D), jnp.float32),
        pltpu.SemaphoreType.DMA,
        pltpu.SemaphoreType.DMA,
        pltpu.SemaphoreType.DMA,
        pltpu.SemaphoreType.DMA,
    ],
)(_sc_agg_body)


def _mm(a, b):
    return jnp.dot(a, b, precision="highest", preferred_element_type=jnp.float32)


BR = 1024


def _tc1_body(agg_ref, cnt_ref, x_ref, wl_ref, wr_ref, b_ref, out_ref):
    i = pl.program_id(0)
    aggf = agg_ref[0] + agg_ref[1]
    cnt = jnp.sum(cnt_ref[...], axis=0)
    rows = i * BR + lax.broadcasted_iota(jnp.int32, (BR, 1), 0)
    cnt = cnt - jnp.where(rows < (EP - E) - NP, 2.0, 1.0)
    inv = 1.0 / jnp.maximum(cnt, 1.0)
    h = _mm(aggf * inv, wl_ref[...]) + _mm(x_ref[...], wr_ref[...]) + b_ref[...]
    out_ref[...] = jnp.maximum(h, 0.0)


_tc1 = pl.pallas_call(
    _tc1_body,
    grid=(NP // BR,),
    in_specs=[
        pl.BlockSpec((2, BR, D), lambda i: (0, i, 0)),
        pl.BlockSpec((32, BR, 1), lambda i: (0, i, 0)),
        pl.BlockSpec((BR, D), lambda i: (i, 0)),
        pl.BlockSpec((D, D), lambda i: (0, 0)),
        pl.BlockSpec((D, D), lambda i: (0, 0)),
        pl.BlockSpec((1, D), lambda i: (0, 0)),
    ],
    out_specs=pl.BlockSpec((BR, D), lambda i: (i, 0)),
    out_shape=jax.ShapeDtypeStruct((NP, D), jnp.float32),
)


def _tc2_body(agg_ref, cnt_ref, h1_ref, batch_ref, wl_ref, wr_ref, b_ref,
              l1w_ref, l1b_ref, l2w_ref, l2b_ref, out_ref,
              pool_scr, cnt_scr):
    i = pl.program_id(0)
    aggf = agg_ref[0] + agg_ref[1]
    cnt = jnp.sum(cnt_ref[...], axis=0)
    rows = i * BR + lax.broadcasted_iota(jnp.int32, (BR, 1), 0)
    cnt = cnt - jnp.where(rows < (EP - E) - NP, 2.0, 1.0)
    inv = 1.0 / jnp.maximum(cnt, 1.0)
    h2 = _mm(aggf * inv, wl_ref[...]) + _mm(h1_ref[...], wr_ref[...]) + b_ref[...]
    h2 = jnp.maximum(h2, 0.0)

    batch_blk = batch_ref[...]
    oh = (batch_blk == lax.broadcasted_iota(jnp.int32, (BR, G), 1))
    oh = oh.astype(jnp.float32)
    part = lax.dot_general(oh, h2, (((0,), (0,)), ((), ())),
                           precision="highest",
                           preferred_element_type=jnp.float32)
    pcnt = jnp.broadcast_to(jnp.sum(oh, axis=0)[:, None], (G, D))

    @pl.when(i == 0)
    def _():
        pool_scr[...] = jnp.zeros((G, D), jnp.float32)
        cnt_scr[...] = jnp.zeros((G, D), jnp.float32)

    pool_scr[...] += part
    cnt_scr[...] += pcnt

    @pl.when(i == NP // BR - 1)
    def _():
        pooled = pool_scr[...] / jnp.maximum(cnt_scr[...], 1.0)
        t = jnp.maximum(_mm(pooled, l1w_ref[...]) + l1b_ref[...], 0.0)
        logits = _mm(t, l2w_ref[...]) + l2b_ref[...]
        m = jnp.max(logits, axis=1, keepdims=True)
        lse = jnp.log(jnp.sum(jnp.exp(logits - m), axis=1, keepdims=True)) + m
        out_ref[...] = logits - lse


_tc2 = pl.pallas_call(
    _tc2_body,
    grid=(NP // BR,),
    in_specs=[
        pl.BlockSpec((2, BR, D), lambda i: (0, i, 0)),
        pl.BlockSpec((32, BR, 1), lambda i: (0, i, 0)),
        pl.BlockSpec((BR, D), lambda i: (i, 0)),
        pl.BlockSpec((BR, 1), lambda i: (i, 0)),
        pl.BlockSpec((D, D), lambda i: (0, 0)),
        pl.BlockSpec((D, D), lambda i: (0, 0)),
        pl.BlockSpec((1, D), lambda i: (0, 0)),
        pl.BlockSpec((D, D), lambda i: (0, 0)),
        pl.BlockSpec((1, D), lambda i: (0, 0)),
        pl.BlockSpec((D, C), lambda i: (0, 0)),
        pl.BlockSpec((1, C), lambda i: (0, 0)),
    ],
    out_specs=pl.BlockSpec((G, C), lambda i: (0, 0)),
    out_shape=jax.ShapeDtypeStruct((G, C), jnp.float32),
    scratch_shapes=[
        pltpu.VMEM((G, D), jnp.float32),
        pltpu.VMEM((G, D), jnp.float32),
    ],
)


@jax.jit
def kernel(x, edge_index, edge_weight, batch,
           conv1_Wl, conv1_Wr, conv1_b,
           conv2_Wl, conv2_Wr, conv2_b,
           lin1_W, lin1_b, lin2_W, lin2_b):
    src = edge_index[0].astype(jnp.int32)
    dst = edge_index[1].astype(jnp.int32)
    w = edge_weight.astype(jnp.float32)
    pad = EP - E
    pad_idx = jnp.arange(pad, dtype=jnp.int32) % NP
    spad = jnp.concatenate([src, pad_idx])
    dpad = jnp.concatenate([dst, pad_idx])
    srcr = spad.reshape(TSB, SB, 128)
    dstr = dpad.reshape(TSB, SB, 128)
    sd3 = jnp.concatenate([srcr, dstr], axis=1)
    w3 = jnp.concatenate([w, jnp.zeros((pad,), jnp.float32)]).reshape(
        TSB, SB * 128)

    xp = jnp.pad(x, ((0, NP - N), (0, 0)))
    zeros2d = jnp.zeros((NP, D), jnp.float32)
    zeros1d = jnp.zeros((NP,), jnp.float32)

    agg1, cnt1 = _sc_agg(xp, sd3, w3, zeros2d, zeros1d)
    cnt1r = cnt1.reshape(32, NP, 1)
    h1 = _tc1(agg1, cnt1r, xp, conv1_Wl, conv1_Wr, conv1_b.reshape(1, D))
    agg2, _ = _sc_agg(h1, sd3, w3, zeros2d, zeros1d)

    batch_p = jnp.concatenate(
        [batch.astype(jnp.int32), jnp.full((NP - N,), G, jnp.int32)]
    ).reshape(NP, 1)
    out = _tc2(agg2, cnt1r, h1, batch_p,
               conv2_Wl, conv2_Wr, conv2_b.reshape(1, D),
               lin1_W, lin1_b.reshape(1, D),
               lin2_W, lin2_b.reshape(1, C))
    return out


# speedup vs baseline: 2.9053x; 1.0827x over previous
import functools

import jax
import jax.numpy as jnp
from jax import lax
from jax.experimental import pallas as pl
from jax.experimental.pallas import tpu as pltpu
from jax.experimental.pallas import tpu_sc as plsc

N = 10000
NP = 10240
E = 640000
EP = 655360
D = 128
G = 64
C = 10
PAD_EDGES = float(EP - E)

SB = 10
TSB = EP // (SB * 128)
NSB0 = TSB // 32
NSB1 = TSB // 32
ROWS_PT = NP // 16


def _sc_agg_body(x_hbm, sd3, w3, zeros2d, zeros1d,
                 agg_out, cnt_out,
                 agg_sh, hist, sd_v, w_v, rows0, rows1,
                 sg0, sg1, ss0, ss1):
    c = lax.axis_index("c")
    s = lax.axis_index("s")
    base_sb = jnp.where(c == 0, s * NSB0, 16 * NSB0 + s * NSB1)
    nsb = jnp.where(c == 0, NSB0, NSB1)
    r = s * ROWS_PT
    pltpu.sync_copy(zeros2d.at[pl.ds(r, ROWS_PT)], agg_sh.at[pl.ds(r, ROWS_PT)])
    pltpu.sync_copy(zeros1d, hist)
    plsc.subcore_barrier()

    bufs = (rows0, rows1)
    gsems = (sg0, sg1)
    ssems = (ss0, ss1)
    ones16 = jnp.ones((16,), jnp.float32)

    def superchunk(k, carry):
        g = base_sb + k
        pltpu.sync_copy(sd3.at[g], sd_v)
        pltpu.sync_copy(w3.at[g], w_v)

        def gather(j):
            return pltpu.async_copy(x_hbm.at[sd_v.at[j]], bufs[j % 2],
                                    gsems[j % 2])

        def scale(j):
            buf = bufs[j % 2]

            def body(g, c2):
                wv = w_v[pl.ds(j * 128 + g * 16, 16)]
                dst16 = sd_v[SB + j, pl.ds(g * 16, 16)]
                plsc.addupdate_scatter(hist, [dst16], ones16)
                for u in range(16):
                    ws = wv[u]
                    e = g * 16 + u
                    for f4 in range(D // 16):
                        sl = buf[e, pl.ds(f4 * 16, 16)]
                        buf[e, pl.ds(f4 * 16, 16)] = sl * ws
                return c2

            lax.fori_loop(0, 8, body, 0)

        def scatter(j):
            return pltpu.async_copy(bufs[j % 2], agg_sh.at[sd_v.at[SB + j]],
                                    ssems[j % 2], add=True)

        gh = {0: gather(0)}
        sh = {}
        for j in range(SB):
            gh[j].wait()
            if j + 1 < SB:
                if j >= 1:
                    sh[j - 1].wait()
                gh[j + 1] = gather(j + 1)
            scale(j)
            sh[j] = scatter(j)
        sh[SB - 2].wait()
        sh[SB - 1].wait()
        return carry

    lax.fori_loop(0, nsb, superchunk, 0)
    plsc.subcore_barrier()
    pltpu.sync_copy(agg_sh.at[pl.ds(r, ROWS_PT)],
                    agg_out.at[c, pl.ds(r, ROWS_PT)])
    pltpu.sync_copy(hist, cnt_out.at[c, s])


_sc_agg = functools.partial(
    pl.kernel,
    mesh=plsc.VectorSubcoreMesh(core_axis_name="c", subcore_axis_name="s"),
    compiler_params=pltpu.CompilerParams(needs_layout_passes=False),
    out_type=(
        jax.ShapeDtypeStruct((2, NP, D), jnp.float32),
        jax.ShapeDtypeStruct((2, 16, NP), jnp.float32),
    ),
    scratch_types=[
        pltpu.VMEM_SHARED((NP, D), jnp.float32),
        pltpu.VMEM((NP,), jnp.float32),
        pltpu.VMEM((2 * SB, 128), jnp.int32),
        pltpu.VMEM((SB * 128,), jnp.float32),
        pltpu.VMEM((128, D), jnp.float32),
        pltpu.VMEM((128, D), jnp.float32),
        pltpu.SemaphoreType.DMA,
        pltpu.SemaphoreType.DMA,
        pltpu.SemaphoreType.DMA,
        pltpu.SemaphoreType.DMA,
    ],
)(_sc_agg_body)


def _mm(a, b):
    return jnp.dot(a, b, preferred_element_type=jnp.float32)


BR = 1024


def _tc1_body(agg_ref, cnt_ref, x_ref, wl_ref, wr_ref, b_ref, out_ref):
    i = pl.program_id(0)
    aggf = agg_ref[0] + agg_ref[1]
    cnt = jnp.sum(cnt_ref[...], axis=0)
    rows = i * BR + lax.broadcasted_iota(jnp.int32, (BR, 1), 0)
    cnt = cnt - jnp.where(rows < (EP - E) - NP, 2.0, 1.0)
    inv = 1.0 / jnp.maximum(cnt, 1.0)
    h = _mm(aggf * inv, wl_ref[...]) + _mm(x_ref[...], wr_ref[...]) + b_ref[...]
    out_ref[...] = jnp.maximum(h, 0.0)


_tc1 = pl.pallas_call(
    _tc1_body,
    grid=(NP // BR,),
    in_specs=[
        pl.BlockSpec((2, BR, D), lambda i: (0, i, 0)),
        pl.BlockSpec((32, BR, 1), lambda i: (0, i, 0)),
        pl.BlockSpec((BR, D), lambda i: (i, 0)),
        pl.BlockSpec((D, D), lambda i: (0, 0)),
        pl.BlockSpec((D, D), lambda i: (0, 0)),
        pl.BlockSpec((1, D), lambda i: (0, 0)),
    ],
    out_specs=pl.BlockSpec((BR, D), lambda i: (i, 0)),
    out_shape=jax.ShapeDtypeStruct((NP, D), jnp.float32),
)


def _tc2_body(agg_ref, cnt_ref, h1_ref, batch_ref, wl_ref, wr_ref, b_ref,
              l1w_ref, l1b_ref, l2w_ref, l2b_ref, out_ref,
              pool_scr, cnt_scr):
    i = pl.program_id(0)
    aggf = agg_ref[0] + agg_ref[1]
    cnt = jnp.sum(cnt_ref[...], axis=0)
    rows = i * BR + lax.broadcasted_iota(jnp.int32, (BR, 1), 0)
    cnt = cnt - jnp.where(rows < (EP - E) - NP, 2.0, 1.0)
    inv = 1.0 / jnp.maximum(cnt, 1.0)
    h2 = _mm(aggf * inv, wl_ref[...]) + _mm(h1_ref[...], wr_ref[...]) + b_ref[...]
    h2 = jnp.maximum(h2, 0.0)

    batch_blk = batch_ref[...]
    oh = (batch_blk == lax.broadcasted_iota(jnp.int32, (BR, G), 1))
    oh = oh.astype(jnp.float32)
    part = lax.dot_general(oh, h2, (((0,), (0,)), ((), ())),
                           preferred_element_type=jnp.float32)
    pcnt = jnp.broadcast_to(jnp.sum(oh, axis=0)[:, None], (G, D))

    @pl.when(i == 0)
    def _():
        pool_scr[...] = jnp.zeros((G, D), jnp.float32)
        cnt_scr[...] = jnp.zeros((G, D), jnp.float32)

    pool_scr[...] += part
    cnt_scr[...] += pcnt

    @pl.when(i == NP // BR - 1)
    def _():
        pooled = pool_scr[...] / jnp.maximum(cnt_scr[...], 1.0)
        t = jnp.maximum(_mm(pooled, l1w_ref[...]) + l1b_ref[...], 0.0)
        logits = _mm(t, l2w_ref[...]) + l2b_ref[...]
        m = jnp.max(logits, axis=1, keepdims=True)
        lse = jnp.log(jnp.sum(jnp.exp(logits - m), axis=1, keepdims=True)) + m
        out_ref[...] = logits - lse


_tc2 = pl.pallas_call(
    _tc2_body,
    grid=(NP // BR,),
    in_specs=[
        pl.BlockSpec((2, BR, D), lambda i: (0, i, 0)),
        pl.BlockSpec((32, BR, 1), lambda i: (0, i, 0)),
        pl.BlockSpec((BR, D), lambda i: (i, 0)),
        pl.BlockSpec((BR, 1), lambda i: (i, 0)),
        pl.BlockSpec((D, D), lambda i: (0, 0)),
        pl.BlockSpec((D, D), lambda i: (0, 0)),
        pl.BlockSpec((1, D), lambda i: (0, 0)),
        pl.BlockSpec((D, D), lambda i: (0, 0)),
        pl.BlockSpec((1, D), lambda i: (0, 0)),
        pl.BlockSpec((D, C), lambda i: (0, 0)),
        pl.BlockSpec((1, C), lambda i: (0, 0)),
    ],
    out_specs=pl.BlockSpec((G, C), lambda i: (0, 0)),
    out_shape=jax.ShapeDtypeStruct((G, C), jnp.float32),
    scratch_shapes=[
        pltpu.VMEM((G, D), jnp.float32),
        pltpu.VMEM((G, D), jnp.float32),
    ],
)


@jax.jit
def kernel(x, edge_index, edge_weight, batch,
           conv1_Wl, conv1_Wr, conv1_b,
           conv2_Wl, conv2_Wr, conv2_b,
           lin1_W, lin1_b, lin2_W, lin2_b):
    src = edge_index[0].astype(jnp.int32)
    dst = edge_index[1].astype(jnp.int32)
    w = edge_weight.astype(jnp.float32)
    pad = EP - E
    pad_idx = jnp.arange(pad, dtype=jnp.int32) % NP
    spad = jnp.concatenate([src, pad_idx])
    dpad = jnp.concatenate([dst, pad_idx])
    srcr = spad.reshape(TSB, SB, 128)
    dstr = dpad.reshape(TSB, SB, 128)
    sd3 = jnp.concatenate([srcr, dstr], axis=1)
    w3 = jnp.concatenate([w, jnp.zeros((pad,), jnp.float32)]).reshape(
        TSB, SB * 128)

    xp = jnp.pad(x, ((0, NP - N), (0, 0)))
    zeros2d = jnp.zeros((NP, D), jnp.float32)
    zeros1d = jnp.zeros((NP,), jnp.float32)

    agg1, cnt1 = _sc_agg(xp, sd3, w3, zeros2d, zeros1d)
    cnt1r = cnt1.reshape(32, NP, 1)
    h1 = _tc1(agg1, cnt1r, xp, conv1_Wl, conv1_Wr, conv1_b.reshape(1, D))
    agg2, _ = _sc_agg(h1, sd3, w3, zeros2d, zeros1d)

    batch_p = jnp.concatenate(
        [batch.astype(jnp.int32), jnp.full((NP - N,), G, jnp.int32)]
    ).reshape(NP, 1)
    out = _tc2(agg2, cnt1r, h1, batch_p,
               conv2_Wl, conv2_Wr, conv2_b.reshape(1, D),
               lin1_W, lin1_b.reshape(1, D),
               lin2_W, lin2_b.reshape(1, C))
    return out


# speedup vs baseline: 2.9227x; 1.0060x over previous
import functools

import jax
import jax.numpy as jnp
from jax import lax
from jax.experimental import pallas as pl
from jax.experimental.pallas import tpu as pltpu
from jax.experimental.pallas import tpu_sc as plsc

N = 10000
NP = 10240
E = 640000
EP = 655360
D = 128
G = 64
C = 10
PAD_SPLIT = (EP - E) - N

SB = 10
TSB = EP // (SB * 128)
NSB = TSB // 32
ROWS_PT = NP // 16


def _sc_agg_body(x_hbm, src3, dst3, w3, zeros2d, zeros1d,
                 agg_out, cnt_out,
                 agg_sh, hist, src_v, dst_v, w_v, rows0, rows1,
                 sg0, sg1, ss0, ss1):
    c = lax.axis_index("c")
    s = lax.axis_index("s")
    wid = s * 2 + c
    r = s * ROWS_PT
    pltpu.sync_copy(zeros2d.at[pl.ds(r, ROWS_PT)], agg_sh.at[pl.ds(r, ROWS_PT)])
    pltpu.sync_copy(zeros1d, hist)
    plsc.subcore_barrier()

    bufs = (rows0, rows1)
    gsems = (sg0, sg1)
    ssems = (ss0, ss1)
    ones16 = jnp.ones((16,), jnp.float32)
    bidx = [jnp.full((16,), u, jnp.int32) for u in range(16)]

    def superchunk(k, carry):
        g = wid * NSB + k
        pltpu.sync_copy(src3.at[g], src_v)
        pltpu.sync_copy(dst3.at[g], dst_v)
        pltpu.sync_copy(w3.at[g], w_v)

        def gather(j):
            return pltpu.async_copy(x_hbm.at[src_v.at[j]], bufs[j % 2],
                                    gsems[j % 2])

        def scale(j):
            buf = bufs[j % 2]

            def body(gg, c2):
                wv = w_v[pl.ds(j * 128 + gg * 16, 16)]
                dst16 = dst_v[j, pl.ds(gg * 16, 16)]
                plsc.addupdate_scatter(hist, [dst16], ones16)
                for u in range(16):
                    wsv = jnp.take_along_axis(wv, bidx[u], axis=0,
                                              mode="promise_in_bounds")
                    e = gg * 16 + u
                    for f4 in range(D // 16):
                        sl = buf[e, pl.ds(f4 * 16, 16)]
                        buf[e, pl.ds(f4 * 16, 16)] = sl * wsv
                return c2

            lax.fori_loop(0, 8, body, 0)

        def scatter(j):
            return pltpu.async_copy(bufs[j % 2], agg_sh.at[dst_v.at[j]],
                                    ssems[j % 2], add=True)

        gh = {0: gather(0)}
        sh = {}
        for j in range(SB):
            gh[j].wait()
            if j + 1 < SB:
                if j >= 1:
                    sh[j - 1].wait()
                gh[j + 1] = gather(j + 1)
            scale(j)
            sh[j] = scatter(j)
        sh[SB - 2].wait()
        sh[SB - 1].wait()
        return carry

    lax.fori_loop(0, NSB, superchunk, 0)
    plsc.subcore_barrier()
    pltpu.sync_copy(agg_sh.at[pl.ds(r, ROWS_PT)],
                    agg_out.at[c, pl.ds(r, ROWS_PT)])
    pltpu.sync_copy(hist, cnt_out.at[c, s])


_sc_agg = functools.partial(
    pl.kernel,
    mesh=plsc.VectorSubcoreMesh(core_axis_name="c", subcore_axis_name="s"),
    compiler_params=pltpu.CompilerParams(needs_layout_passes=False),
    out_type=(
        jax.ShapeDtypeStruct((2, NP, D), jnp.float32),
        jax.ShapeDtypeStruct((2, 16, NP), jnp.float32),
    ),
    scratch_types=[
        pltpu.VMEM_SHARED((NP, D), jnp.float32),
        pltpu.VMEM((NP,), jnp.float32),
        pltpu.VMEM((SB, 128), jnp.int32),
        pltpu.VMEM((SB, 128), jnp.int32),
        pltpu.VMEM((SB * 128,), jnp.float32),
        pltpu.VMEM((128, D), jnp.float32),
        pltpu.VMEM((128, D), jnp.float32),
        pltpu.SemaphoreType.DMA,
        pltpu.SemaphoreType.DMA,
        pltpu.SemaphoreType.DMA,
        pltpu.SemaphoreType.DMA,
    ],
)(_sc_agg_body)


def _mm(a, b):
    return jnp.dot(a, b, preferred_element_type=jnp.float32)


BR = 1000


def _cnt_from(cnt_ref, i):
    cnt = jnp.sum(cnt_ref[...], axis=0)
    rows = i * BR + lax.broadcasted_iota(jnp.int32, (BR, 1), 0)
    cnt = cnt - jnp.where(rows < PAD_SPLIT, 2.0, 1.0)
    return 1.0 / jnp.maximum(cnt, 1.0)


def _tc1_body(agg_ref, cnt_ref, x_ref, wl_ref, wr_ref, b_ref, out_ref):
    i = pl.program_id(0)
    aggf = agg_ref[0] + agg_ref[1]
    inv = _cnt_from(cnt_ref, i)
    h = _mm(aggf * inv, wl_ref[...]) + _mm(x_ref[...], wr_ref[...]) + b_ref[...]
    out_ref[...] = jnp.maximum(h, 0.0)


_tc1 = pl.pallas_call(
    _tc1_body,
    grid=(N // BR,),
    in_specs=[
        pl.BlockSpec((2, BR, D), lambda i: (0, i, 0)),
        pl.BlockSpec((32, BR, 1), lambda i: (0, i, 0)),
        pl.BlockSpec((BR, D), lambda i: (i, 0)),
        pl.BlockSpec((D, D), lambda i: (0, 0)),
        pl.BlockSpec((D, D), lambda i: (0, 0)),
        pl.BlockSpec((1, D), lambda i: (0, 0)),
    ],
    out_specs=pl.BlockSpec((BR, D), lambda i: (i, 0)),
    out_shape=jax.ShapeDtypeStruct((N, D), jnp.float32),
)


def _tc2_body(agg_ref, cnt_ref, h1_ref, batch_ref, wl_ref, wr_ref, b_ref,
              l1w_ref, l1b_ref, l2w_ref, l2b_ref, out_ref,
              pool_scr, cnt_scr):
    i = pl.program_id(0)
    aggf = agg_ref[0] + agg_ref[1]
    inv = _cnt_from(cnt_ref, i)
    h2 = _mm(aggf * inv, wl_ref[...]) + _mm(h1_ref[...], wr_ref[...]) + b_ref[...]
    h2 = jnp.maximum(h2, 0.0)

    batch_blk = batch_ref[...]
    oh = (batch_blk == lax.broadcasted_iota(jnp.int32, (BR, G), 1))
    oh = oh.astype(jnp.float32)
    part = lax.dot_general(oh, h2, (((0,), (0,)), ((), ())),
                           preferred_element_type=jnp.float32)
    pcnt = jnp.broadcast_to(jnp.sum(oh, axis=0)[:, None], (G, D))

    @pl.when(i == 0)
    def _():
        pool_scr[...] = jnp.zeros((G, D), jnp.float32)
        cnt_scr[...] = jnp.zeros((G, D), jnp.float32)

    pool_scr[...] += part
    cnt_scr[...] += pcnt

    @pl.when(i == N // BR - 1)
    def _():
        pooled = pool_scr[...] / jnp.maximum(cnt_scr[...], 1.0)
        t = jnp.maximum(_mm(pooled, l1w_ref[...]) + l1b_ref[...], 0.0)
        logits = _mm(t, l2w_ref[...]) + l2b_ref[...]
        m = jnp.max(logits, axis=1, keepdims=True)
        lse = jnp.log(jnp.sum(jnp.exp(logits - m), axis=1, keepdims=True)) + m
        out_ref[...] = logits - lse


_tc2 = pl.pallas_call(
    _tc2_body,
    grid=(N // BR,),
    in_specs=[
        pl.BlockSpec((2, BR, D), lambda i: (0, i, 0)),
        pl.BlockSpec((32, BR, 1), lambda i: (0, i, 0)),
        pl.BlockSpec((BR, D), lambda i: (i, 0)),
        pl.BlockSpec((BR, 1), lambda i: (i, 0)),
        pl.BlockSpec((D, D), lambda i: (0, 0)),
        pl.BlockSpec((D, D), lambda i: (0, 0)),
        pl.BlockSpec((1, D), lambda i: (0, 0)),
        pl.BlockSpec((D, D), lambda i: (0, 0)),
        pl.BlockSpec((1, D), lambda i: (0, 0)),
        pl.BlockSpec((D, C), lambda i: (0, 0)),
        pl.BlockSpec((1, C), lambda i: (0, 0)),
    ],
    out_specs=pl.BlockSpec((G, C), lambda i: (0, 0)),
    out_shape=jax.ShapeDtypeStruct((G, C), jnp.float32),
    scratch_shapes=[
        pltpu.VMEM((G, D), jnp.float32),
        pltpu.VMEM((G, D), jnp.float32),
    ],
)


@jax.jit
def kernel(x, edge_index, edge_weight, batch,
           conv1_Wl, conv1_Wr, conv1_b,
           conv2_Wl, conv2_Wr, conv2_b,
           lin1_W, lin1_b, lin2_W, lin2_b):
    src = edge_index[0].astype(jnp.int32)
    dst = edge_index[1].astype(jnp.int32)
    w = edge_weight.astype(jnp.float32)
    pad = EP - E
    pad_idx = jnp.arange(pad, dtype=jnp.int32) % N
    src3 = jnp.concatenate([src, pad_idx]).reshape(TSB, SB, 128)
    dst3 = jnp.concatenate([dst, pad_idx]).reshape(TSB, SB, 128)
    w3 = jnp.concatenate([w, jnp.zeros((pad,), jnp.float32)]).reshape(
        TSB, SB * 128)

    zeros2d = jnp.zeros((NP, D), jnp.float32)
    zeros1d = jnp.zeros((NP,), jnp.float32)

    agg1, cnt1 = _sc_agg(x, src3, dst3, w3, zeros2d, zeros1d)
    cnt1r = cnt1.reshape(32, NP, 1)
    h1 = _tc1(agg1, cnt1r, x, conv1_Wl, conv1_Wr, conv1_b.reshape(1, D))
    agg2, _ = _sc_agg(h1, src3, dst3, w3, zeros2d, zeros1d)

    out = _tc2(agg2, cnt1r, h1, batch.astype(jnp.int32).reshape(N, 1),
               conv2_Wl, conv2_Wr, conv2_b.reshape(1, D),
               lin1_W, lin1_b.reshape(1, D),
               lin2_W, lin2_b.reshape(1, C))
    return out
